# baseline (speedup 1.0000x reference)
import functools

import jax
import jax.numpy as jnp
from jax import lax
from jax.experimental import pallas as pl
from jax.experimental.pallas import tpu as pltpu
from jax.experimental.pallas import tpu_sc as plsc

N_NODES = 10000
D_FEAT = 128
NC = 2
NS = 16
NW = NC * NS
LANES = 16
E_CHUNK = 128
NBUF = 2
ROWS_PER_TILE = 624
ROWS_TAIL = N_NODES - NS * ROWS_PER_TILE


def _sc_partials(n_chunks):
    mesh = plsc.VectorSubcoreMesh(
        core_axis_name="c", subcore_axis_name="s", num_cores=NC, num_subcores=NS
    )

    @functools.partial(
        pl.kernel,
        out_type=jax.ShapeDtypeStruct((NC, N_NODES, D_FEAT), jnp.float32),
        mesh=mesh,
        scratch_types=[
            pltpu.VMEM((NBUF, 3, E_CHUNK), jnp.int32),
            pltpu.VMEM((NBUF, E_CHUNK, D_FEAT), jnp.float32),
            pltpu.VMEM_SHARED((N_NODES, D_FEAT), jnp.float32),
            pltpu.SemaphoreType.DMA,
            pltpu.SemaphoreType.DMA,
            pltpu.SemaphoreType.DMA,
            pltpu.SemaphoreType.DMA,
        ],
    )
    def k(edges_hbm, x_hbm, out_hbm, eb, rows, acc, si0, si1, sr0, sr1):
        cid = lax.axis_index("c")
        sid = lax.axis_index("s")
        tile_id = cid * NS + sid
        sem_i = (si0, si1)
        sem_r = (sr0, sr1)

        def zero_row(r, _):
            for j in range(D_FEAT // LANES):
                rows[0, r, pl.ds(j * LANES, LANES)] = jnp.zeros((LANES,), jnp.float32)
            return _

        lax.fori_loop(0, E_CHUNK, zero_row, None)

        row_base = sid * ROWS_PER_TILE
        n_full = ROWS_PER_TILE // E_CHUNK
        rem = ROWS_PER_TILE - n_full * E_CHUNK
        for kk in range(n_full):
            pltpu.sync_copy(rows.at[0], acc.at[pl.ds(row_base + kk * E_CHUNK, E_CHUNK)])
        pltpu.sync_copy(
            rows.at[0, pl.ds(0, rem)],
            acc.at[pl.ds(row_base + n_full * E_CHUNK, rem)],
        )

        @pl.when(sid == NS - 1)
        def _zero_tail():
            pltpu.sync_copy(
                rows.at[0, pl.ds(0, ROWS_TAIL)],
                acc.at[pl.ds(NS * ROWS_PER_TILE, ROWS_TAIL)],
            )

        plsc.subcore_barrier()

        pltpu.async_copy(edges_hbm.at[tile_id, 0], eb.at[0], si0)
        pltpu.async_copy(edges_hbm.at[tile_id, 1], eb.at[1], si1)
        pltpu.make_async_copy(edges_hbm.at[tile_id, 0], eb.at[0], si0).wait()
        pltpu.async_copy(x_hbm.at[eb.at[0, 0]], rows.at[0], sr0)

        def pair_body(h, _):
            for b in range(NBUF):
                g = h * NBUF + b
                bn = (b + 1) % NBUF
                ebb = eb.at[b]
                rows_b = rows.at[b]

                pltpu.make_async_copy(x_hbm.at[ebb.at[0]], rows_b, sem_r[b]).wait()

                @pl.when(g + 1 < n_chunks)
                def _next_gather():
                    pltpu.make_async_copy(
                        edges_hbm.at[tile_id, g + 1], eb.at[bn], sem_i[bn]
                    ).wait()
                    pltpu.async_copy(x_hbm.at[eb.at[bn, 0]], rows.at[bn], sem_r[bn])

                def scale_16(t, _2):
                    e0 = t * LANES
                    a16 = lax.bitcast_convert_type(
                        ebb[2, pl.ds(e0, LANES)], jnp.float32
                    )
                    for l in range(LANES):
                        a = a16[l]
                        for j in range(D_FEAT // LANES):
                            sl = pl.ds(j * LANES, LANES)
                            rows_b[e0 + l, sl] = rows_b[e0 + l, sl] * a
                    return _2

                lax.fori_loop(0, E_CHUNK // LANES, scale_16, None)
                pltpu.sync_copy(rows_b, acc.at[ebb.at[1]], add=True)

                @pl.when(g + NBUF < n_chunks)
                def _next_desc():
                    pltpu.async_copy(edges_hbm.at[tile_id, g + NBUF], ebb, sem_i[b])

            return _

        lax.fori_loop(0, n_chunks // NBUF, pair_body, None)
        plsc.subcore_barrier()

        for kk in range(n_full):
            r0 = row_base + kk * E_CHUNK
            pltpu.sync_copy(acc.at[pl.ds(r0, E_CHUNK)], rows.at[0])
            pltpu.sync_copy(rows.at[0], out_hbm.at[cid, pl.ds(r0, E_CHUNK)])
        r0 = row_base + n_full * E_CHUNK
        pltpu.sync_copy(acc.at[pl.ds(r0, rem)], rows.at[0, pl.ds(0, rem)])
        pltpu.sync_copy(rows.at[0, pl.ds(0, rem)], out_hbm.at[cid, pl.ds(r0, rem)])

        @pl.when(sid == NS - 1)
        def _pub_tail():
            t0 = NS * ROWS_PER_TILE
            pltpu.sync_copy(acc.at[pl.ds(t0, ROWS_TAIL)], rows.at[0, pl.ds(0, ROWS_TAIL)])
            pltpu.sync_copy(rows.at[0, pl.ds(0, ROWS_TAIL)], out_hbm.at[cid, pl.ds(t0, ROWS_TAIL)])

    return k


def _combine_body(p_ref, o_ref):
    o_ref[...] = p_ref[0] + p_ref[1]


def _combine(partials):
    rows_blk = 1000
    return pl.pallas_call(
        _combine_body,
        out_shape=jax.ShapeDtypeStruct((N_NODES, D_FEAT), jnp.float32),
        grid=(N_NODES // rows_blk,),
        in_specs=[pl.BlockSpec((NC, rows_blk, D_FEAT), lambda i: (0, i, 0))],
        out_specs=pl.BlockSpec((rows_blk, D_FEAT), lambda i: (i, 0)),
    )(partials)


@jax.jit
def kernel(x, edge_index, adj_values):
    n_edges = edge_index.shape[1]
    n_chunks = -(-n_edges // (NW * E_CHUNK))
    n_chunks = -(-n_chunks // NBUF) * NBUF
    e_pad = n_chunks * E_CHUNK * NW

    dst = edge_index[0].astype(jnp.int32)
    src = edge_index[1].astype(jnp.int32)
    adj = lax.bitcast_convert_type(adj_values.astype(jnp.float32), jnp.int32)
    pad = e_pad - n_edges
    if pad:
        dst = jnp.concatenate([dst, jnp.zeros((pad,), jnp.int32)])
        src = jnp.concatenate([src, jnp.zeros((pad,), jnp.int32)])
        adj = jnp.concatenate([adj, jnp.zeros((pad,), jnp.int32)])
    edges = jnp.stack(
        [
            src.reshape(NW, n_chunks, E_CHUNK),
            dst.reshape(NW, n_chunks, E_CHUNK),
            adj.reshape(NW, n_chunks, E_CHUNK),
        ],
        axis=2,
    )

    partials = _sc_partials(n_chunks)(edges, x)
    return _combine(partials)


# speedup vs baseline: 1.0076x; 1.0076x over previous
import functools

import jax
import jax.numpy as jnp
from jax import lax
from jax.experimental import pallas as pl
from jax.experimental.pallas import tpu as pltpu
from jax.experimental.pallas import tpu_sc as plsc

N_NODES = 10000
D_FEAT = 128
NC = 2
NS = 16
NW = NC * NS
LANES = 16
E_CHUNK = 128
NBUF = 2
ERING = 4
ROWS_PER_TILE = 624
ROWS_TAIL = N_NODES - NS * ROWS_PER_TILE


def _sc_partials(n_chunks):
    mesh = plsc.VectorSubcoreMesh(
        core_axis_name="c", subcore_axis_name="s", num_cores=NC, num_subcores=NS
    )

    @functools.partial(
        pl.kernel,
        out_type=jax.ShapeDtypeStruct((NC, N_NODES, D_FEAT), jnp.float32),
        mesh=mesh,
        scratch_types=[
            pltpu.VMEM((ERING, 3, E_CHUNK), jnp.int32),
            pltpu.VMEM((NBUF, E_CHUNK, D_FEAT), jnp.float32),
            pltpu.VMEM_SHARED((N_NODES, D_FEAT), jnp.float32),
            pltpu.SemaphoreType.DMA,
            pltpu.SemaphoreType.DMA,
            pltpu.SemaphoreType.DMA,
            pltpu.SemaphoreType.DMA,
            pltpu.SemaphoreType.DMA,
            pltpu.SemaphoreType.DMA,
            pltpu.SemaphoreType.DMA,
            pltpu.SemaphoreType.DMA,
        ],
    )
    def k(edges_hbm, x_hbm, out_hbm, eb, rows, acc,
          si0, si1, si2, si3, sr0, sr1, ss0, ss1):
        cid = lax.axis_index("c")
        sid = lax.axis_index("s")
        tile_id = cid * NS + sid
        sem_i = (si0, si1, si2, si3)
        sem_r = (sr0, sr1)
        sem_s = (ss0, ss1)

        def zero_row(r, _):
            for j in range(D_FEAT // LANES):
                rows[0, r, pl.ds(j * LANES, LANES)] = jnp.zeros((LANES,), jnp.float32)
            return _

        lax.fori_loop(0, E_CHUNK, zero_row, None)

        row_base = sid * ROWS_PER_TILE
        n_full = ROWS_PER_TILE // E_CHUNK
        rem = ROWS_PER_TILE - n_full * E_CHUNK
        for kk in range(n_full):
            pltpu.sync_copy(rows.at[0], acc.at[pl.ds(row_base + kk * E_CHUNK, E_CHUNK)])
        pltpu.sync_copy(
            rows.at[0, pl.ds(0, rem)],
            acc.at[pl.ds(row_base + n_full * E_CHUNK, rem)],
        )

        @pl.when(sid == NS - 1)
        def _zero_tail():
            pltpu.sync_copy(
                rows.at[0, pl.ds(0, ROWS_TAIL)],
                acc.at[pl.ds(NS * ROWS_PER_TILE, ROWS_TAIL)],
            )

        plsc.subcore_barrier()

        pltpu.async_copy(edges_hbm.at[tile_id, 0], eb.at[0], si0)
        pltpu.async_copy(edges_hbm.at[tile_id, 1], eb.at[1], si1)
        pltpu.make_async_copy(edges_hbm.at[tile_id, 0], eb.at[0], si0).wait()
        pltpu.async_copy(x_hbm.at[eb.at[0, 0]], rows.at[0], sr0)

        def ring_body(h, _):
            for b in range(ERING):
                g = h * ERING + b
                rb = b % NBUF
                rbn = (b + 1) % NBUF
                es = b
                esn = (b + 1) % ERING
                esp = (b - 1) % ERING
                es2 = (b + 2) % ERING
                ebb = eb.at[es]
                rows_b = rows.at[rb]

                pltpu.make_async_copy(x_hbm.at[ebb.at[0]], rows_b, sem_r[rb]).wait()

                @pl.when(g + 2 < n_chunks)
                def _next_desc():
                    pltpu.async_copy(edges_hbm.at[tile_id, g + 2], eb.at[es2], sem_i[es2])

                @pl.when(g + 1 < n_chunks)
                def _next_gather():
                    pltpu.make_async_copy(
                        edges_hbm.at[tile_id, g + 1], eb.at[esn], sem_i[esn]
                    ).wait()

                    @pl.when(g >= 1)
                    def _drain_prev_scatter():
                        pltpu.make_async_copy(
                            rows.at[rbn], acc.at[eb.at[esp, 1]], sem_s[rbn]
                        ).wait()

                    pltpu.async_copy(x_hbm.at[eb.at[esn, 0]], rows.at[rbn], sem_r[rbn])

                def scale_16(t, _2):
                    e0 = t * LANES
                    a16 = lax.bitcast_convert_type(
                        ebb[2, pl.ds(e0, LANES)], jnp.float32
                    )
                    for l in range(LANES):
                        a = a16[l]
                        for j in range(D_FEAT // LANES):
                            sl = pl.ds(j * LANES, LANES)
                            rows_b[e0 + l, sl] = rows_b[e0 + l, sl] * a
                    return _2

                lax.fori_loop(0, E_CHUNK // LANES, scale_16, None)
                pltpu.async_copy(rows_b, acc.at[ebb.at[1]], sem_s[rb], add=True)

            return _

        lax.fori_loop(0, n_chunks // ERING, ring_body, None)

        for g in (n_chunks - 2, n_chunks - 1):
            pltpu.make_async_copy(
                rows.at[g % NBUF], acc.at[eb.at[g % ERING, 1]], sem_s[g % NBUF]
            ).wait()
        plsc.subcore_barrier()

        for kk in range(n_full):
            r0 = row_base + kk * E_CHUNK
            pltpu.sync_copy(acc.at[pl.ds(r0, E_CHUNK)], rows.at[0])
            pltpu.sync_copy(rows.at[0], out_hbm.at[cid, pl.ds(r0, E_CHUNK)])
        r0 = row_base + n_full * E_CHUNK
        pltpu.sync_copy(acc.at[pl.ds(r0, rem)], rows.at[0, pl.ds(0, rem)])
        pltpu.sync_copy(rows.at[0, pl.ds(0, rem)], out_hbm.at[cid, pl.ds(r0, rem)])

        @pl.when(sid == NS - 1)
        def _pub_tail():
            t0 = NS * ROWS_PER_TILE
            pltpu.sync_copy(acc.at[pl.ds(t0, ROWS_TAIL)], rows.at[0, pl.ds(0, ROWS_TAIL)])
            pltpu.sync_copy(rows.at[0, pl.ds(0, ROWS_TAIL)], out_hbm.at[cid, pl.ds(t0, ROWS_TAIL)])

    return k


def _combine_body(p_ref, o_ref):
    o_ref[...] = p_ref[0] + p_ref[1]


def _combine(partials):
    rows_blk = 1000
    return pl.pallas_call(
        _combine_body,
        out_shape=jax.ShapeDtypeStruct((N_NODES, D_FEAT), jnp.float32),
        grid=(N_NODES // rows_blk,),
        in_specs=[pl.BlockSpec((NC, rows_blk, D_FEAT), lambda i: (0, i, 0))],
        out_specs=pl.BlockSpec((rows_blk, D_FEAT), lambda i: (i, 0)),
    )(partials)


@jax.jit
def kernel(x, edge_index, adj_values):
    n_edges = edge_index.shape[1]
    n_chunks = -(-n_edges // (NW * E_CHUNK))
    n_chunks = -(-n_chunks // ERING) * ERING
    e_pad = n_chunks * E_CHUNK * NW

    dst = edge_index[0].astype(jnp.int32)
    src = edge_index[1].astype(jnp.int32)
    adj = lax.bitcast_convert_type(adj_values.astype(jnp.float32), jnp.int32)
    pad = e_pad - n_edges
    if pad:
        dst = jnp.concatenate([dst, jnp.zeros((pad,), jnp.int32)])
        src = jnp.concatenate([src, jnp.zeros((pad,), jnp.int32)])
        adj = jnp.concatenate([adj, jnp.zeros((pad,), jnp.int32)])
    edges = jnp.stack(
        [
            src.reshape(NW, n_chunks, E_CHUNK),
            dst.reshape(NW, n_chunks, E_CHUNK),
            adj.reshape(NW, n_chunks, E_CHUNK),
        ],
        axis=2,
    )

    partials = _sc_partials(n_chunks)(edges, x)
    return _combine(partials)
